# baseline (device time: 67062 ns/iter reference)
import jax
import jax.numpy as jnp
from jax import lax
from jax.experimental import pallas as pl
from jax.experimental.pallas import tpu as pltpu

N_DEV = 8
B_LOC = 2
SQ = 128
SKV = 128
HQ = 32
DH = 64
H_LOC = HQ // N_DEV
D_MODEL = 512
HD_LOC = H_LOC * DH


def kernel(x, Wq, K_ext, V_ext, Wo):
    my = lax.axis_index("i")

    kb = lax.dynamic_slice_in_dim(K_ext, my * B_LOC, B_LOC, axis=0)
    vb = lax.dynamic_slice_in_dim(V_ext, my * B_LOC, B_LOC, axis=0)
    k4 = jnp.transpose(kb, (0, 2, 1, 3)).astype(jnp.bfloat16)
    v4 = jnp.transpose(vb, (0, 2, 1, 3)).astype(jnp.bfloat16)
    x2 = x.reshape(B_LOC * SQ, D_MODEL).astype(jnp.bfloat16)
    wq = Wq.astype(jnp.bfloat16)
    wo = Wo.astype(jnp.bfloat16)

    def body(x_ref, wq_ref, k_ref, v_ref, wo_ref, out_ref,
             comm_wq, comm_wo, ctx_scr,
             send_wq, recv_wq, send_wo, recv_wo):
        my_pos = lax.axis_index("i")
        left = lax.rem(my_pos + N_DEV - 1, N_DEV)
        right = lax.rem(my_pos + 1, N_DEV)

        barrier = pltpu.get_barrier_semaphore()
        for nbr in (left, right):
            pl.semaphore_signal(barrier, inc=1, device_id=(nbr,),
                                device_id_type=pl.DeviceIdType.MESH)
        pl.semaphore_wait(barrier, 2)

        comm_wq[0] = wq_ref[...]
        comm_wo[0] = wo_ref[...]

        qi = lax.broadcasted_iota(jnp.int32, (SQ, SKV), 0) // 64
        ki = lax.broadcasted_iota(jnp.int32, (SQ, SKV), 1) // 64
        mask = ki <= qi

        for h in range(N_DEV):
            slot = h % 2
            nxt = (h + 1) % 2
            if h < N_DEV - 1:
                rdma_wq = pltpu.make_async_remote_copy(
                    src_ref=comm_wq.at[slot], dst_ref=comm_wq.at[nxt],
                    send_sem=send_wq.at[slot], recv_sem=recv_wq.at[nxt],
                    device_id=(right,), device_id_type=pl.DeviceIdType.MESH)
                rdma_wo = pltpu.make_async_remote_copy(
                    src_ref=comm_wo.at[slot], dst_ref=comm_wo.at[nxt],
                    send_sem=send_wo.at[slot], recv_sem=recv_wo.at[nxt],
                    device_id=(right,), device_id_type=pl.DeviceIdType.MESH)
                rdma_wq.start()
                rdma_wo.start()

            grp = lax.rem(my_pos - h + N_DEV, N_DEV)

            q_all = jax.lax.dot_general(
                x_ref[...], comm_wq[slot],
                (((1,), (0,)), ((), ())),
                preferred_element_type=jnp.float32,
            ).astype(jnp.bfloat16)

            for b in range(B_LOC):
                for hh in range(H_LOC):
                    q = q_all[b * SQ:(b + 1) * SQ, hh * DH:(hh + 1) * DH]
                    kk = k_ref[b, grp * H_LOC + hh]
                    s = jax.lax.dot_general(
                        q, kk, (((1,), (1,)), ((), ())),
                        preferred_element_type=jnp.float32) * 0.125
                    s = jnp.where(mask, s, -1e9)
                    m = jnp.max(s, axis=1, keepdims=True)
                    w = jnp.exp(s - m)
                    w = w / jnp.sum(w, axis=1, keepdims=True)
                    vv = v_ref[b, grp * H_LOC + hh]
                    ctx = jax.lax.dot_general(
                        w.astype(jnp.bfloat16), vv,
                        (((1,), (0,)), ((), ())),
                        preferred_element_type=jnp.float32)
                    ctx_scr[b * SQ:(b + 1) * SQ, hh * DH:(hh + 1) * DH] = (
                        ctx.astype(jnp.bfloat16))

            contrib = jax.lax.dot_general(
                ctx_scr[...], comm_wo[slot],
                (((1,), (0,)), ((), ())),
                preferred_element_type=jnp.float32)
            if h == 0:
                out_ref[...] = contrib
            else:
                out_ref[...] = out_ref[...] + contrib

            if h < N_DEV - 1:
                rdma_wq.wait()
                rdma_wo.wait()

    out = pl.pallas_call(
        body,
        out_shape=jax.ShapeDtypeStruct((B_LOC * SQ, D_MODEL), jnp.float32),
        in_specs=[pl.BlockSpec(memory_space=pltpu.VMEM)] * 5,
        out_specs=pl.BlockSpec(memory_space=pltpu.VMEM),
        scratch_shapes=[
            pltpu.VMEM((2, D_MODEL, HD_LOC), jnp.bfloat16),
            pltpu.VMEM((2, HD_LOC, D_MODEL), jnp.bfloat16),
            pltpu.VMEM((B_LOC * SQ, HD_LOC), jnp.bfloat16),
            pltpu.SemaphoreType.DMA((2,)),
            pltpu.SemaphoreType.DMA((2,)),
            pltpu.SemaphoreType.DMA((2,)),
            pltpu.SemaphoreType.DMA((2,)),
        ],
        compiler_params=pltpu.CompilerParams(collective_id=0),
    )(x2, wq, k4, v4, wo)

    return out.reshape(B_LOC, SQ, D_MODEL)


# device time: 45083 ns/iter; 1.4875x vs baseline; 1.4875x over previous
import jax
import jax.numpy as jnp
from jax import lax
from jax.experimental import pallas as pl
from jax.experimental.pallas import tpu as pltpu

N_DEV = 8
B_LOC = 2
SQ = 128
SKV = 128
HQ = 32
DH = 64
H_LOC = HQ // N_DEV
D_MODEL = 512
HD_LOC = H_LOC * DH

CW_HOPS = 4
CCW_HOPS = 3


def kernel(x, Wq, K_ext, V_ext, Wo):
    my = lax.axis_index("i")

    kb = lax.dynamic_slice_in_dim(K_ext, my * B_LOC, B_LOC, axis=0)
    vb = lax.dynamic_slice_in_dim(V_ext, my * B_LOC, B_LOC, axis=0)
    k4 = jnp.transpose(kb, (0, 2, 1, 3)).astype(jnp.bfloat16)
    v4 = jnp.transpose(vb, (0, 2, 1, 3)).astype(jnp.bfloat16)
    x2 = x.reshape(B_LOC * SQ, D_MODEL).astype(jnp.bfloat16)
    wq = Wq.astype(jnp.bfloat16)
    wo = Wo.astype(jnp.bfloat16)

    def body(x_ref, wq_ref, k_ref, v_ref, wo_ref, out_ref,
             g_wq, g_wo, ctx_scr,
             send_wq, recv_wq, send_wo, recv_wo):
        my_pos = lax.axis_index("i")
        left = lax.rem(my_pos + N_DEV - 1, N_DEV)
        right = lax.rem(my_pos + 1, N_DEV)

        barrier = pltpu.get_barrier_semaphore()
        for nbr in (left, right):
            pl.semaphore_signal(barrier, inc=1, device_id=(nbr,),
                                device_id_type=pl.DeviceIdType.MESH)
        pl.semaphore_wait(barrier, 2)

        def send_pair(wq_src, wo_src, dst_slot, sem_idx, target):
            swq = pltpu.make_async_remote_copy(
                src_ref=wq_src, dst_ref=g_wq.at[dst_slot],
                send_sem=send_wq.at[sem_idx], recv_sem=recv_wq.at[dst_slot],
                device_id=(target,), device_id_type=pl.DeviceIdType.MESH)
            swo = pltpu.make_async_remote_copy(
                src_ref=wo_src, dst_ref=g_wo.at[dst_slot],
                send_sem=send_wo.at[sem_idx], recv_sem=recv_wo.at[dst_slot],
                device_id=(target,), device_id_type=pl.DeviceIdType.MESH)
            swq.start()
            swo.start()
            return swq, swo

        def wait_recv(slot):
            rwq = pltpu.make_async_remote_copy(
                src_ref=wq_ref, dst_ref=g_wq.at[slot],
                send_sem=send_wq.at[0], recv_sem=recv_wq.at[slot],
                device_id=(left,), device_id_type=pl.DeviceIdType.MESH)
            rwo = pltpu.make_async_remote_copy(
                src_ref=wo_ref, dst_ref=g_wo.at[slot],
                send_sem=send_wo.at[0], recv_sem=recv_wo.at[slot],
                device_id=(left,), device_id_type=pl.DeviceIdType.MESH)
            rwq.wait_recv()
            rwo.wait_recv()

        qi = lax.broadcasted_iota(jnp.int32, (SQ, SKV), 0) // 64
        ki = lax.broadcasted_iota(jnp.int32, (SQ, SKV), 1) // 64
        mask = ki <= qi

        def compute(d, wq_slot, wo_slot, first):
            grp = lax.rem(my_pos - d + N_DEV, N_DEV)
            q_all = jax.lax.dot_general(
                x_ref[...], wq_slot,
                (((1,), (0,)), ((), ())),
                preferred_element_type=jnp.float32,
            ).astype(jnp.bfloat16)
            for b in range(B_LOC):
                for hh in range(H_LOC):
                    q = q_all[b * SQ:(b + 1) * SQ, hh * DH:(hh + 1) * DH]
                    kk = k_ref[b, grp * H_LOC + hh]
                    s = jax.lax.dot_general(
                        q, kk, (((1,), (1,)), ((), ())),
                        preferred_element_type=jnp.float32) * 0.125
                    s = jnp.where(mask, s, -1e9)
                    m = jnp.max(s, axis=1, keepdims=True)
                    w = jnp.exp(s - m)
                    w = w / jnp.sum(w, axis=1, keepdims=True)
                    vv = v_ref[b, grp * H_LOC + hh]
                    ctx = jax.lax.dot_general(
                        w.astype(jnp.bfloat16), vv,
                        (((1,), (0,)), ((), ())),
                        preferred_element_type=jnp.float32)
                    ctx_scr[b * SQ:(b + 1) * SQ, hh * DH:(hh + 1) * DH] = (
                        ctx.astype(jnp.bfloat16))
            contrib = jax.lax.dot_general(
                ctx_scr[...], wo_slot,
                (((1,), (0,)), ((), ())),
                preferred_element_type=jnp.float32)
            if first:
                out_ref[...] = contrib
            else:
                out_ref[...] = out_ref[...] + contrib

        sends = []
        sends += send_pair(wq_ref, wo_ref, 0, 0, right)
        sends += send_pair(wq_ref, wo_ref, 6, 4, left)
        compute(0, wq_ref[...], wo_ref[...], first=True)

        for r in range(1, 4):
            cw = r - 1
            ccw = 7 - r
            wait_recv(cw)
            if r < CW_HOPS:
                sends += send_pair(g_wq.at[cw], g_wo.at[cw], cw + 1, r, right)
            wait_recv(ccw)
            if r < CCW_HOPS:
                sends += send_pair(g_wq.at[ccw], g_wo.at[ccw], ccw - 1,
                                   4 + r, left)
            compute(r, g_wq[cw], g_wo[cw], first=False)
            compute(8 - r, g_wq[ccw], g_wo[ccw], first=False)

        wait_recv(3)
        compute(4, g_wq[3], g_wo[3], first=False)

        for s in sends:
            s.wait_send()

    out = pl.pallas_call(
        body,
        out_shape=jax.ShapeDtypeStruct((B_LOC * SQ, D_MODEL), jnp.float32),
        in_specs=[pl.BlockSpec(memory_space=pltpu.VMEM)] * 5,
        out_specs=pl.BlockSpec(memory_space=pltpu.VMEM),
        scratch_shapes=[
            pltpu.VMEM((7, D_MODEL, HD_LOC), jnp.bfloat16),
            pltpu.VMEM((7, HD_LOC, D_MODEL), jnp.bfloat16),
            pltpu.VMEM((B_LOC * SQ, HD_LOC), jnp.bfloat16),
            pltpu.SemaphoreType.DMA((7,)),
            pltpu.SemaphoreType.DMA((7,)),
            pltpu.SemaphoreType.DMA((7,)),
            pltpu.SemaphoreType.DMA((7,)),
        ],
        compiler_params=pltpu.CompilerParams(collective_id=0),
    )(x2, wq, k4, v4, wo)

    return out.reshape(B_LOC, SQ, D_MODEL)


# device time: 39247 ns/iter; 1.7087x vs baseline; 1.1487x over previous
import jax
import jax.numpy as jnp
from jax import lax
from jax.experimental import pallas as pl
from jax.experimental.pallas import tpu as pltpu

N_DEV = 8
B_LOC = 2
SQ = 128
SKV = 128
HQ = 32
DH = 64
H_LOC = HQ // N_DEV
D_MODEL = 512
HD_LOC = H_LOC * DH

CW_HOPS = 4
CCW_HOPS = 3


def kernel(x, Wq, K_ext, V_ext, Wo):
    my = lax.axis_index("i")

    kb = lax.dynamic_slice_in_dim(K_ext, my * B_LOC, B_LOC, axis=0)
    vb = lax.dynamic_slice_in_dim(V_ext, my * B_LOC, B_LOC, axis=0)
    k4 = jnp.transpose(kb, (0, 2, 1, 3)).astype(jnp.bfloat16)
    v4 = jnp.transpose(vb, (0, 2, 1, 3)).astype(jnp.bfloat16)
    x2 = x.reshape(B_LOC * SQ, D_MODEL).astype(jnp.bfloat16)

    sq = jnp.maximum(jnp.max(jnp.abs(Wq), axis=0, keepdims=True), 1e-30) / 127.0
    qwq = jnp.round(Wq / sq).astype(jnp.int8)
    so = jnp.maximum(jnp.max(jnp.abs(Wo), axis=0, keepdims=True), 1e-30) / 127.0
    qwo = jnp.round(Wo / so).astype(jnp.int8)
    sq = sq.astype(jnp.float32)
    so = so.astype(jnp.float32)

    def body(x_ref, qwq_ref, sq_ref, qwo_ref, so_ref, k_ref, v_ref, out_ref,
             g_qwq, g_sq, g_qwo, g_so, ctx_scr,
             s_qwq, r_qwq, s_sq, r_sq, s_qwo, r_qwo, s_so, r_so):
        my_pos = lax.axis_index("i")
        left = lax.rem(my_pos + N_DEV - 1, N_DEV)
        right = lax.rem(my_pos + 1, N_DEV)

        barrier = pltpu.get_barrier_semaphore()
        for nbr in (left, right):
            pl.semaphore_signal(barrier, inc=1, device_id=(nbr,),
                                device_id_type=pl.DeviceIdType.MESH)
        pl.semaphore_wait(barrier, 2)

        def chunk_rdmas(srcs, dst_slot, sem_idx, target):
            descs = []
            for src, g, ssem, rsem in (
                (srcs[0], g_qwq, s_qwq, r_qwq),
                (srcs[1], g_sq, s_sq, r_sq),
                (srcs[2], g_qwo, s_qwo, r_qwo),
                (srcs[3], g_so, s_so, r_so),
            ):
                descs.append(pltpu.make_async_remote_copy(
                    src_ref=src, dst_ref=g.at[dst_slot],
                    send_sem=ssem.at[sem_idx], recv_sem=rsem.at[dst_slot],
                    device_id=(target,), device_id_type=pl.DeviceIdType.MESH))
            return descs

        local_srcs = (qwq_ref, sq_ref, qwo_ref, so_ref)

        def slot_srcs(slot):
            return (g_qwq.at[slot], g_sq.at[slot], g_qwo.at[slot],
                    g_so.at[slot])

        def send_chunk(srcs, dst_slot, sem_idx, target):
            descs = chunk_rdmas(srcs, dst_slot, sem_idx, target)
            for d_ in descs:
                d_.start()
            return descs

        def wait_recv(slot):
            for d_ in chunk_rdmas(local_srcs, slot, 0, left):
                d_.wait_recv()

        qi = lax.broadcasted_iota(jnp.int32, (SQ, SKV), 0) // 64
        ki = lax.broadcasted_iota(jnp.int32, (SQ, SKV), 1) // 64
        mask = ki <= qi

        def compute(d, qwq_v, sq_v, qwo_v, so_v, first):
            grp = lax.rem(my_pos - d + N_DEV, N_DEV)
            q_all = jax.lax.dot_general(
                x_ref[...], qwq_v.astype(jnp.bfloat16),
                (((1,), (0,)), ((), ())),
                preferred_element_type=jnp.float32,
            ) * sq_v
            q_all = q_all.astype(jnp.bfloat16)
            for b in range(B_LOC):
                for hh in range(H_LOC):
                    q = q_all[b * SQ:(b + 1) * SQ, hh * DH:(hh + 1) * DH]
                    kk = k_ref[b, grp * H_LOC + hh]
                    s = jax.lax.dot_general(
                        q, kk, (((1,), (1,)), ((), ())),
                        preferred_element_type=jnp.float32) * 0.125
                    s = jnp.where(mask, s, -1e9)
                    m = jnp.max(s, axis=1, keepdims=True)
                    w = jnp.exp(s - m)
                    w = w / jnp.sum(w, axis=1, keepdims=True)
                    vv = v_ref[b, grp * H_LOC + hh]
                    ctx = jax.lax.dot_general(
                        w.astype(jnp.bfloat16), vv,
                        (((1,), (0,)), ((), ())),
                        preferred_element_type=jnp.float32)
                    ctx_scr[b * SQ:(b + 1) * SQ, hh * DH:(hh + 1) * DH] = (
                        ctx.astype(jnp.bfloat16))
            contrib = jax.lax.dot_general(
                ctx_scr[...], qwo_v.astype(jnp.bfloat16),
                (((1,), (0,)), ((), ())),
                preferred_element_type=jnp.float32) * so_v
            if first:
                out_ref[...] = contrib
            else:
                out_ref[...] = out_ref[...] + contrib

        sends = []
        sends += send_chunk(local_srcs, 0, 0, right)
        sends += send_chunk(local_srcs, 6, 4, left)
        compute(0, qwq_ref[...], sq_ref[...], qwo_ref[...], so_ref[...],
                first=True)

        for r in range(1, 4):
            cw = r - 1
            ccw = 7 - r
            wait_recv(cw)
            if r < CW_HOPS:
                sends += send_chunk(slot_srcs(cw), cw + 1, r, right)
            wait_recv(ccw)
            if r < CCW_HOPS:
                sends += send_chunk(slot_srcs(ccw), ccw - 1, 4 + r, left)
            compute(r, g_qwq[cw], g_sq[cw], g_qwo[cw], g_so[cw],
                    first=False)
            compute(8 - r, g_qwq[ccw], g_sq[ccw], g_qwo[ccw], g_so[ccw],
                    first=False)

        wait_recv(3)
        compute(4, g_qwq[3], g_sq[3], g_qwo[3], g_so[3], first=False)

        for s_ in sends:
            s_.wait_send()

    out = pl.pallas_call(
        body,
        out_shape=jax.ShapeDtypeStruct((B_LOC * SQ, D_MODEL), jnp.float32),
        in_specs=[pl.BlockSpec(memory_space=pltpu.VMEM)] * 7,
        out_specs=pl.BlockSpec(memory_space=pltpu.VMEM),
        scratch_shapes=[
            pltpu.VMEM((7, D_MODEL, HD_LOC), jnp.int8),
            pltpu.VMEM((7, 1, HD_LOC), jnp.float32),
            pltpu.VMEM((7, HD_LOC, D_MODEL), jnp.int8),
            pltpu.VMEM((7, 1, D_MODEL), jnp.float32),
            pltpu.VMEM((B_LOC * SQ, HD_LOC), jnp.bfloat16),
            pltpu.SemaphoreType.DMA((7,)),
            pltpu.SemaphoreType.DMA((7,)),
            pltpu.SemaphoreType.DMA((7,)),
            pltpu.SemaphoreType.DMA((7,)),
            pltpu.SemaphoreType.DMA((7,)),
            pltpu.SemaphoreType.DMA((7,)),
            pltpu.SemaphoreType.DMA((7,)),
            pltpu.SemaphoreType.DMA((7,)),
        ],
        compiler_params=pltpu.CompilerParams(collective_id=0),
    )(x2, qwq, sq, qwo, so, k4, v4)

    return out.reshape(B_LOC, SQ, D_MODEL)


# device time: 34332 ns/iter; 1.9533x vs baseline; 1.1432x over previous
import jax
import jax.numpy as jnp
from jax import lax
from jax.experimental import pallas as pl
from jax.experimental.pallas import tpu as pltpu

N_DEV = 8
B_LOC = 2
SQ = 128
SKV = 128
HQ = 32
DH = 64
H_LOC = HQ // N_DEV
D_MODEL = 512
HD_LOC = H_LOC * DH
SBLK = H_LOC * SKV
VAUG = HD_LOC + 128

CW_HOPS = 4
CCW_HOPS = 3


def kernel(x, Wq, K_ext, V_ext, Wo):
    my = lax.axis_index("i")

    kb = lax.dynamic_slice_in_dim(K_ext, my * B_LOC, B_LOC, axis=0)
    vb = lax.dynamic_slice_in_dim(V_ext, my * B_LOC, B_LOC, axis=0)
    kT = jnp.transpose(kb, (0, 2, 3, 1)).astype(jnp.bfloat16)
    v4 = jnp.transpose(vb, (0, 2, 1, 3)).astype(jnp.bfloat16)
    x2 = x.reshape(B_LOC * SQ, D_MODEL).astype(jnp.bfloat16)

    sq = jnp.maximum(jnp.max(jnp.abs(Wq), axis=0, keepdims=True), 1e-30) / 127.0
    qwq = jnp.round(Wq * (127.0 / jnp.maximum(jnp.max(jnp.abs(Wq), axis=0,
                                                      keepdims=True), 1e-30))
                    ).astype(jnp.int8)
    sq = (sq * 0.125).astype(jnp.float32)
    ao = jnp.maximum(jnp.max(jnp.abs(Wo), axis=0, keepdims=True), 1e-30)
    qwo = jnp.round(Wo * (127.0 / ao)).astype(jnp.int8)
    so = (ao / 127.0).astype(jnp.float32)

    def body(x_ref, qwq_ref, sq_ref, qwo_ref, so_ref, k_ref, v_ref, out_ref,
             g_qwq, g_sq, g_qwo, g_so, ctx_scr, kblk, vblk,
             s_qwq, r_qwq, s_sq, r_sq, s_qwo, r_qwo, s_so, r_so):
        my_pos = lax.axis_index("i")
        left = lax.rem(my_pos + N_DEV - 1, N_DEV)
        right = lax.rem(my_pos + 1, N_DEV)

        barrier = pltpu.get_barrier_semaphore()
        for nbr in (left, right):
            pl.semaphore_signal(barrier, inc=1, device_id=(nbr,),
                                device_id_type=pl.DeviceIdType.MESH)
        pl.semaphore_wait(barrier, 2)

        def chunk_rdmas(srcs, dst_slot, sem_idx, target):
            descs = []
            for src, g, ssem, rsem in (
                (srcs[0], g_qwq, s_qwq, r_qwq),
                (srcs[1], g_sq, s_sq, r_sq),
                (srcs[2], g_qwo, s_qwo, r_qwo),
                (srcs[3], g_so, s_so, r_so),
            ):
                descs.append(pltpu.make_async_remote_copy(
                    src_ref=src, dst_ref=g.at[dst_slot],
                    send_sem=ssem.at[sem_idx], recv_sem=rsem.at[dst_slot],
                    device_id=(target,), device_id_type=pl.DeviceIdType.MESH))
            return descs

        local_srcs = (qwq_ref, sq_ref, qwo_ref, so_ref)

        def slot_srcs(slot):
            return (g_qwq.at[slot], g_sq.at[slot], g_qwo.at[slot],
                    g_so.at[slot])

        def send_chunk(srcs, dst_slot, sem_idx, target):
            descs = chunk_rdmas(srcs, dst_slot, sem_idx, target)
            for d_ in descs:
                d_.start()
            return descs

        def wait_recv(slot):
            for d_ in chunk_rdmas(local_srcs, slot, 0, left):
                d_.wait_recv()

        for b in range(B_LOC):
            kblk[b] = jnp.zeros((HD_LOC, SBLK), jnp.bfloat16)
            rows = lax.broadcasted_iota(jnp.int32, (SBLK, 128), 0) // SKV
            cols = lax.broadcasted_iota(jnp.int32, (SBLK, 128), 1)
            ones_pat = (rows == cols).astype(jnp.bfloat16)
            vblk[b] = jnp.concatenate(
                [jnp.zeros((SBLK, HD_LOC), jnp.bfloat16), ones_pat], axis=1)

        qblk = lax.broadcasted_iota(jnp.int32, (SQ, SBLK), 0) // 64
        kblk_id = (lax.broadcasted_iota(jnp.int32, (SQ, SBLK), 1) % SKV) // 64
        mask = kblk_id <= qblk

        def compute(d, qwq_v, sq_v, qwo_v, so_v, first):
            grp = lax.rem(my_pos - d + N_DEV, N_DEV)
            g4 = grp * H_LOC
            q_all = (jax.lax.dot_general(
                x_ref[...], qwq_v.astype(jnp.bfloat16),
                (((1,), (0,)), ((), ())),
                preferred_element_type=jnp.float32,
            ) * sq_v).astype(jnp.bfloat16)
            for b in range(B_LOC):
                for hh in range(H_LOC):
                    kblk[b, hh * DH:(hh + 1) * DH,
                         hh * SKV:(hh + 1) * SKV] = k_ref[b, g4 + hh]
                    vblk[b, hh * SKV:(hh + 1) * SKV,
                         hh * DH:(hh + 1) * DH] = v_ref[b, g4 + hh]
            for b in range(B_LOC):
                s = jax.lax.dot_general(
                    q_all[b * SQ:(b + 1) * SQ, :], kblk[b],
                    (((1,), (0,)), ((), ())),
                    preferred_element_type=jnp.float32)
                w = jnp.where(mask, jnp.exp(s), 0.0).astype(jnp.bfloat16)
                aug = jax.lax.dot_general(
                    w, vblk[b],
                    (((1,), (0,)), ((), ())),
                    preferred_element_type=jnp.float32)
                rec = 1.0 / aug[:, HD_LOC:HD_LOC + H_LOC]
                scale = jnp.broadcast_to(
                    rec[:, :, None], (SQ, H_LOC, DH)).reshape(SQ, HD_LOC)
                ctx_scr[b * SQ:(b + 1) * SQ, :] = (
                    aug[:, :HD_LOC] * scale).astype(jnp.bfloat16)
            contrib = jax.lax.dot_general(
                ctx_scr[...], qwo_v.astype(jnp.bfloat16),
                (((1,), (0,)), ((), ())),
                preferred_element_type=jnp.float32) * so_v
            if first:
                out_ref[...] = contrib
            else:
                out_ref[...] = out_ref[...] + contrib

        sends = []
        sends += send_chunk(local_srcs, 0, 0, right)
        sends += send_chunk(local_srcs, 6, 4, left)
        compute(0, qwq_ref[...], sq_ref[...], qwo_ref[...], so_ref[...],
                first=True)

        for r in range(1, 4):
            cw = r - 1
            ccw = 7 - r
            wait_recv(cw)
            if r < CW_HOPS:
                sends += send_chunk(slot_srcs(cw), cw + 1, r, right)
            wait_recv(ccw)
            if r < CCW_HOPS:
                sends += send_chunk(slot_srcs(ccw), ccw - 1, 4 + r, left)
            compute(r, g_qwq[cw], g_sq[cw], g_qwo[cw], g_so[cw],
                    first=False)
            compute(8 - r, g_qwq[ccw], g_sq[ccw], g_qwo[ccw], g_so[ccw],
                    first=False)

        wait_recv(3)
        compute(4, g_qwq[3], g_sq[3], g_qwo[3], g_so[3], first=False)

        for s_ in sends:
            s_.wait_send()

    out = pl.pallas_call(
        body,
        out_shape=jax.ShapeDtypeStruct((B_LOC * SQ, D_MODEL), jnp.float32),
        in_specs=[pl.BlockSpec(memory_space=pltpu.VMEM)] * 7,
        out_specs=pl.BlockSpec(memory_space=pltpu.VMEM),
        scratch_shapes=[
            pltpu.VMEM((7, D_MODEL, HD_LOC), jnp.int8),
            pltpu.VMEM((7, 1, HD_LOC), jnp.float32),
            pltpu.VMEM((7, HD_LOC, D_MODEL), jnp.int8),
            pltpu.VMEM((7, 1, D_MODEL), jnp.float32),
            pltpu.VMEM((B_LOC * SQ, HD_LOC), jnp.bfloat16),
            pltpu.VMEM((B_LOC, HD_LOC, SBLK), jnp.bfloat16),
            pltpu.VMEM((B_LOC, SBLK, VAUG), jnp.bfloat16),
            pltpu.SemaphoreType.DMA((7,)),
            pltpu.SemaphoreType.DMA((7,)),
            pltpu.SemaphoreType.DMA((7,)),
            pltpu.SemaphoreType.DMA((7,)),
            pltpu.SemaphoreType.DMA((7,)),
            pltpu.SemaphoreType.DMA((7,)),
            pltpu.SemaphoreType.DMA((7,)),
            pltpu.SemaphoreType.DMA((7,)),
        ],
        compiler_params=pltpu.CompilerParams(collective_id=0),
    )(x2, qwq, sq, qwo, so, kT, v4)

    return out.reshape(B_LOC, SQ, D_MODEL)


# device time: 32115 ns/iter; 2.0882x vs baseline; 1.0690x over previous
import jax
import jax.numpy as jnp
from jax import lax
from jax.experimental import pallas as pl
from jax.experimental.pallas import tpu as pltpu

N_DEV = 8
B_LOC = 2
SQ = 128
SKV = 128
HQ = 32
DH = 64
H_LOC = HQ // N_DEV
D_MODEL = 512
HD_LOC = H_LOC * DH
SBLK = H_LOC * SKV
VAUG = HD_LOC + 128

CW_HOPS = 4
CCW_HOPS = 3

LOCAL = 7


def kernel(x, Wq, K_ext, V_ext, Wo):
    my = lax.axis_index("i")

    kf = lax.dynamic_slice_in_dim(K_ext, my * B_LOC, B_LOC, axis=0)
    kf = kf.reshape(B_LOC, SKV, HQ * DH).astype(jnp.bfloat16)
    vf = lax.dynamic_slice_in_dim(V_ext, my * B_LOC, B_LOC, axis=0)
    vf = vf.reshape(B_LOC, SKV, HQ * DH).astype(jnp.bfloat16)
    x2 = x.reshape(B_LOC * SQ, D_MODEL)

    def body(x_ref, wq_ref, wo_ref, k_ref, v_ref, out_ref,
             g_qwq, g_sq, g_qwo, g_so, xb, ctx_scr, kblkT, vblk,
             s_qwq, r_qwq, s_sq, r_sq, s_qwo, r_qwo, s_so, r_so):
        my_pos = lax.axis_index("i")
        left = lax.rem(my_pos + N_DEV - 1, N_DEV)
        right = lax.rem(my_pos + 1, N_DEV)

        barrier = pltpu.get_barrier_semaphore()
        for nbr in (left, right):
            pl.semaphore_signal(barrier, inc=1, device_id=(nbr,),
                                device_id_type=pl.DeviceIdType.MESH)
        pl.semaphore_wait(barrier, 2)

        wq = wq_ref[...]
        aq = jnp.maximum(jnp.max(jnp.abs(wq), axis=0, keepdims=True), 1e-30)
        g_qwq[LOCAL] = jnp.round(wq * (127.0 / aq)).astype(jnp.int8)
        g_sq[LOCAL] = aq * (0.125 / 127.0)
        wo = wo_ref[...]
        ao = jnp.maximum(jnp.max(jnp.abs(wo), axis=0, keepdims=True), 1e-30)
        g_qwo[LOCAL] = jnp.round(wo * (127.0 / ao)).astype(jnp.int8)
        g_so[LOCAL] = ao * (1.0 / 127.0)

        def chunk_rdmas(src_slot, dst_slot, sem_idx, target):
            descs = []
            for g, ssem, rsem in ((g_qwq, s_qwq, r_qwq),
                                  (g_sq, s_sq, r_sq),
                                  (g_qwo, s_qwo, r_qwo),
                                  (g_so, s_so, r_so)):
                descs.append(pltpu.make_async_remote_copy(
                    src_ref=g.at[src_slot], dst_ref=g.at[dst_slot],
                    send_sem=ssem.at[sem_idx], recv_sem=rsem.at[dst_slot],
                    device_id=(target,), device_id_type=pl.DeviceIdType.MESH))
            return descs

        def send_chunk(src_slot, dst_slot, sem_idx, target):
            descs = chunk_rdmas(src_slot, dst_slot, sem_idx, target)
            for d_ in descs:
                d_.start()
            return descs

        def wait_recv(slot):
            for d_ in chunk_rdmas(LOCAL, slot, 0, left):
                d_.wait_recv()

        for b in range(B_LOC):
            kblkT[b] = jnp.zeros((SBLK, HD_LOC), jnp.bfloat16)
            rows = lax.broadcasted_iota(jnp.int32, (SBLK, 128), 0) // SKV
            cols = lax.broadcasted_iota(jnp.int32, (SBLK, 128), 1)
            ones_pat = (rows == cols).astype(jnp.bfloat16)
            vblk[b] = jnp.concatenate(
                [jnp.zeros((SBLK, HD_LOC), jnp.bfloat16), ones_pat], axis=1)

        qblk = lax.broadcasted_iota(jnp.int32, (SQ, SBLK), 0) // 64
        kblk_id = (lax.broadcasted_iota(jnp.int32, (SQ, SBLK), 1) % SKV) // 64
        mask = kblk_id <= qblk

        def compute(d, slot, first):
            grp = lax.rem(my_pos - d + N_DEV, N_DEV)
            q_all = (jax.lax.dot_general(
                xb[...], g_qwq[slot].astype(jnp.bfloat16),
                (((1,), (0,)), ((), ())),
                preferred_element_type=jnp.float32,
            ) * g_sq[slot]).astype(jnp.bfloat16)
            for b in range(B_LOC):
                kg = k_ref[b, :, pl.ds(grp * HD_LOC, HD_LOC)]
                vg = v_ref[b, :, pl.ds(grp * HD_LOC, HD_LOC)]
                for hh in range(H_LOC):
                    kblkT[b, hh * SKV:(hh + 1) * SKV,
                          hh * DH:(hh + 1) * DH] = kg[:, hh * DH:(hh + 1) * DH]
                    vblk[b, hh * SKV:(hh + 1) * SKV,
                         hh * DH:(hh + 1) * DH] = vg[:, hh * DH:(hh + 1) * DH]
            for b in range(B_LOC):
                s = jax.lax.dot_general(
                    q_all[b * SQ:(b + 1) * SQ, :], kblkT[b],
                    (((1,), (1,)), ((), ())),
                    preferred_element_type=jnp.float32)
                w = jnp.where(mask, jnp.exp(s), 0.0).astype(jnp.bfloat16)
                aug = jax.lax.dot_general(
                    w, vblk[b],
                    (((1,), (0,)), ((), ())),
                    preferred_element_type=jnp.float32)
                rec = 1.0 / aug[:, HD_LOC:HD_LOC + H_LOC]
                scale = jnp.broadcast_to(
                    rec[:, :, None], (SQ, H_LOC, DH)).reshape(SQ, HD_LOC)
                ctx_scr[b * SQ:(b + 1) * SQ, :] = (
                    aug[:, :HD_LOC] * scale).astype(jnp.bfloat16)
            contrib = jax.lax.dot_general(
                ctx_scr[...], g_qwo[slot].astype(jnp.bfloat16),
                (((1,), (0,)), ((), ())),
                preferred_element_type=jnp.float32) * g_so[slot]
            if first:
                out_ref[...] = contrib
            else:
                out_ref[...] = out_ref[...] + contrib

        sends = []
        sends += send_chunk(LOCAL, 0, 0, right)
        sends += send_chunk(LOCAL, 6, 4, left)
        xb[...] = x_ref[...].astype(jnp.bfloat16)
        compute(0, LOCAL, first=True)

        for r in range(1, 4):
            cw = r - 1
            ccw = 7 - r
            wait_recv(cw)
            if r < CW_HOPS:
                sends += send_chunk(cw, cw + 1, r, right)
            wait_recv(ccw)
            if r < CCW_HOPS:
                sends += send_chunk(ccw, ccw - 1, 4 + r, left)
            compute(r, cw, first=False)
            compute(8 - r, ccw, first=False)

        wait_recv(3)
        compute(4, 3, first=False)

        for s_ in sends:
            s_.wait_send()

    out = pl.pallas_call(
        body,
        out_shape=jax.ShapeDtypeStruct((B_LOC * SQ, D_MODEL), jnp.float32),
        in_specs=[pl.BlockSpec(memory_space=pltpu.VMEM)] * 5,
        out_specs=pl.BlockSpec(memory_space=pltpu.VMEM),
        scratch_shapes=[
            pltpu.VMEM((8, D_MODEL, HD_LOC), jnp.int8),
            pltpu.VMEM((8, 1, HD_LOC), jnp.float32),
            pltpu.VMEM((8, HD_LOC, D_MODEL), jnp.int8),
            pltpu.VMEM((8, 1, D_MODEL), jnp.float32),
            pltpu.VMEM((B_LOC * SQ, D_MODEL), jnp.bfloat16),
            pltpu.VMEM((B_LOC * SQ, HD_LOC), jnp.bfloat16),
            pltpu.VMEM((B_LOC, SBLK, HD_LOC), jnp.bfloat16),
            pltpu.VMEM((B_LOC, SBLK, VAUG), jnp.bfloat16),
            pltpu.SemaphoreType.DMA((8,)),
            pltpu.SemaphoreType.DMA((8,)),
            pltpu.SemaphoreType.DMA((8,)),
            pltpu.SemaphoreType.DMA((8,)),
            pltpu.SemaphoreType.DMA((8,)),
            pltpu.SemaphoreType.DMA((8,)),
            pltpu.SemaphoreType.DMA((8,)),
            pltpu.SemaphoreType.DMA((8,)),
        ],
        compiler_params=pltpu.CompilerParams(collective_id=0),
    )(x2, Wq, Wo, kf, vf)

    return out.reshape(B_LOC, SQ, D_MODEL)
